# baseline (speedup 1.0000x reference)
import functools

import jax
import jax.numpy as jnp
from jax import lax
from jax.experimental import pallas as pl
from jax.experimental.pallas import tpu as pltpu
from jax.experimental.pallas import tpu_sc as plsc

NC = 2
NS = 16
NW = NC * NS
LANE = 128
RING = 2
HALVES = 2


def _tc_mm1(x_pad, W1, npad, d):
    grid = npad // 256

    def body(x_ref, w_ref, o_ref):
        o_ref[...] = jnp.dot(x_ref[...], w_ref[...],
                             preferred_element_type=jnp.float32)

    return pl.pallas_call(
        body,
        grid=(grid,),
        in_specs=[
            pl.BlockSpec((256, d), lambda r: (r, 0)),
            pl.BlockSpec((d, d), lambda r: (0, 0)),
        ],
        out_specs=pl.BlockSpec((256, d), lambda r: (r, 0)),
        out_shape=jax.ShapeDtypeStruct((npad, d), jnp.float32),
    )(x_pad, W1)


def _sc_hist(dst3, npad, chunks):
    rows_per_tile = npad // NS
    zc = rows_per_tile // LANE
    mesh = plsc.VectorSubcoreMesh(core_axis_name="c", subcore_axis_name="s",
                                  num_cores=NC, num_subcores=NS)

    @functools.partial(
        pl.kernel,
        out_type=jax.ShapeDtypeStruct((NC, npad, 16), jnp.float32),
        mesh=mesh,
        scratch_types=[
            pltpu.VMEM((chunks, LANE), jnp.int32),
            pltpu.VMEM((LANE, 16), jnp.float32),
            pltpu.VMEM((LANE, 16), jnp.float32),
            pltpu.VMEM_SHARED((npad, 16), jnp.float32),
        ],
    )
    def hist(dst_hbm, out, dst_v, zb_v, ones_v, shared):
        c = lax.axis_index("c")
        s = lax.axis_index("s")
        wid = c * NS + s
        pltpu.sync_copy(dst_hbm.at[wid], dst_v)

        def init_body(i, _):
            zb_v[i, :] = jnp.zeros((16,), jnp.float32)
            ones_v[i, :] = jnp.ones((16,), jnp.float32)
            return _
        lax.fori_loop(0, LANE, init_body, None)
        for k in range(zc):
            pltpu.sync_copy(
                zb_v, shared.at[pl.ds(s * rows_per_tile + k * LANE, LANE)])
        plsc.subcore_barrier()

        def body(j, _):
            pltpu.sync_copy(ones_v, shared.at[dst_v.at[j]], add=True)
            return _
        lax.fori_loop(0, chunks, body, None)
        plsc.subcore_barrier()
        pltpu.sync_copy(shared.at[pl.ds(s * rows_per_tile, rows_per_tile)],
                        out.at[c, pl.ds(s * rows_per_tile, rows_per_tile)])

    return hist(dst3)


def _sc_agg(table, src3, dst3, npad, chunks, d):
    rows_per_tile = npad // NS
    zc = rows_per_tile // LANE
    mesh = plsc.VectorSubcoreMesh(core_axis_name="c", subcore_axis_name="s",
                                  num_cores=NC, num_subcores=NS)

    hchunks = chunks // HALVES

    @functools.partial(
        pl.kernel,
        out_type=jax.ShapeDtypeStruct((NC, npad, d), jnp.float32),
        mesh=mesh,
        scratch_types=[
            pltpu.VMEM((hchunks, LANE), jnp.int32),
            pltpu.VMEM((hchunks, LANE), jnp.int32),
            pltpu.VMEM((RING, LANE, d), jnp.float32),
            pltpu.VMEM_SHARED((npad, d), jnp.float32),
            pltpu.SemaphoreType.DMA,
            pltpu.SemaphoreType.DMA,
        ],
    )
    def agg(table_hbm, src_hbm, dst_hbm, out, src_v, dst_v, rows_v, shared,
            gsem0, gsem1):
        c = lax.axis_index("c")
        s = lax.axis_index("s")
        wid = c * NS + s

        def zero_body(i, _):
            for j in range(d // 16):
                rows_v[0, i, pl.ds(j * 16, 16)] = jnp.zeros((16,), jnp.float32)
            return _
        lax.fori_loop(0, LANE, zero_body, None)
        for k in range(zc):
            pltpu.sync_copy(rows_v.at[0],
                            shared.at[pl.ds(s * rows_per_tile + k * LANE,
                                            LANE)])
        plsc.subcore_barrier()

        def half(h, _):
            pltpu.sync_copy(src_hbm.at[wid, pl.ds(h * hchunks, hchunks)],
                            src_v)
            pltpu.sync_copy(dst_hbm.at[wid, pl.ds(h * hchunks, hchunks)],
                            dst_v)

            gsems = [gsem0, gsem1]

            def group(g, _):
                j = g * RING
                gds = [
                    pltpu.async_copy(table_hbm.at[src_v.at[j + b]],
                                     rows_v.at[b], gsems[b])
                    for b in range(RING)
                ]
                for b in range(RING):
                    gds[b].wait()
                    pltpu.sync_copy(rows_v.at[b], shared.at[dst_v.at[j + b]],
                                    add=True)
                return _
            lax.fori_loop(0, hchunks // RING, group, None)
            return _
        lax.fori_loop(0, HALVES, half, None)
        plsc.subcore_barrier()
        pltpu.sync_copy(shared.at[pl.ds(s * rows_per_tile, rows_per_tile)],
                        out.at[c, pl.ds(s * rows_per_tile, rows_per_tile)])

    return agg(table, src3, dst3)


def _tc_scale1(h1, histp, npad, d):
    grid = npad // 256

    def body(h_ref, hp_ref, s_ref, dis_ref):
        deg = 1.0 + hp_ref[0] + hp_ref[1]
        dis = lax.rsqrt(deg)
        dis_ref[...] = dis
        s_ref[...] = h_ref[...] * dis[:, 0:1]

    return pl.pallas_call(
        body,
        grid=(grid,),
        in_specs=[
            pl.BlockSpec((256, d), lambda r: (r, 0)),
            pl.BlockSpec((NC, 256, 16), lambda r: (0, r, 0)),
        ],
        out_specs=[
            pl.BlockSpec((256, d), lambda r: (r, 0)),
            pl.BlockSpec((256, 16), lambda r: (r, 0)),
        ],
        out_shape=[
            jax.ShapeDtypeStruct((npad, d), jnp.float32),
            jax.ShapeDtypeStruct((npad, 16), jnp.float32),
        ],
    )(h1, histp)


def _tc_l2(agg1, scaled1, dis, b1, W2, npad, d, d2):
    grid = npad // 256

    def body(a_ref, s1_ref, dis_ref, b1_ref, w2_ref, o_ref):
        di = dis_ref[:, 0:1]
        u = di * (a_ref[0] + a_ref[1] + s1_ref[...]) + b1_ref[...]
        h = jnp.maximum(u, 0.0)
        s2 = di * jnp.dot(h, w2_ref[...], preferred_element_type=jnp.float32)
        o_ref[...] = jnp.concatenate(
            [s2, jnp.zeros((s2.shape[0], d - d2), jnp.float32)], axis=1)

    return pl.pallas_call(
        body,
        grid=(grid,),
        in_specs=[
            pl.BlockSpec((NC, 256, d), lambda r: (0, r, 0)),
            pl.BlockSpec((256, d), lambda r: (r, 0)),
            pl.BlockSpec((256, 16), lambda r: (r, 0)),
            pl.BlockSpec((1, d), lambda r: (0, 0)),
            pl.BlockSpec((d, d2), lambda r: (0, 0)),
        ],
        out_specs=pl.BlockSpec((256, d), lambda r: (r, 0)),
        out_shape=jax.ShapeDtypeStruct((npad, d), jnp.float32),
    )(agg1, scaled1, dis, b1, W2)


def _tc_out(agg2, scaled2, dis, b2, npad, d, d2):
    grid = npad // 256

    def body(a_ref, s2_ref, dis_ref, b2_ref, lp_ref, lg_ref):
        di = dis_ref[:, 0:1]
        full = di * (a_ref[0] + a_ref[1] + s2_ref[...])
        logits = full[:, :d2] + b2_ref[...]
        lg_ref[...] = logits
        m = jnp.max(logits, axis=1, keepdims=True)
        e = jnp.exp(logits - m)
        lse = m + jnp.log(jnp.sum(e, axis=1, keepdims=True))
        lp_ref[...] = logits - lse

    return pl.pallas_call(
        body,
        grid=(grid,),
        in_specs=[
            pl.BlockSpec((NC, 256, d), lambda r: (0, r, 0)),
            pl.BlockSpec((256, d), lambda r: (r, 0)),
            pl.BlockSpec((256, 16), lambda r: (r, 0)),
            pl.BlockSpec((1, d2), lambda r: (0, 0)),
        ],
        out_specs=[
            pl.BlockSpec((256, d2), lambda r: (r, 0)),
            pl.BlockSpec((256, d2), lambda r: (r, 0)),
        ],
        out_shape=[
            jax.ShapeDtypeStruct((npad, d2), jnp.float32),
            jax.ShapeDtypeStruct((npad, d2), jnp.float32),
        ],
    )(agg2, scaled2, dis, b2)


def kernel(x, edge_index, W1, b1, W2, b2):
    n, d = x.shape
    d2 = W2.shape[1]
    e = edge_index.shape[1]

    npad = -(-n // 2048) * 2048
    chunks = -(-e // (NW * LANE))
    gran = RING * HALVES
    chunks = -(-chunks // gran) * gran
    epad = NW * chunks * LANE

    x_pad = jnp.pad(x, ((0, npad - n), (0, 0)))
    pad_idx = jnp.full((epad - e,), npad - 1, jnp.int32)
    src3 = jnp.concatenate([edge_index[0].astype(jnp.int32), pad_idx]
                           ).reshape(NW, chunks, LANE)
    dst3 = jnp.concatenate([edge_index[1].astype(jnp.int32), pad_idx]
                           ).reshape(NW, chunks, LANE)
    b1r = b1.reshape(1, d)
    b2r = b2.reshape(1, d2)

    h1 = _tc_mm1(x_pad, W1, npad, d)
    histp = _sc_hist(dst3, npad, chunks)
    scaled1, dis = _tc_scale1(h1, histp, npad, d)
    agg1 = _sc_agg(scaled1, src3, dst3, npad, chunks, d)
    scaled2 = _tc_l2(agg1, scaled1, dis, b1r, W2, npad, d, d2)
    agg2 = _sc_agg(scaled2, src3, dst3, npad, chunks, d)
    log_probs, logits = _tc_out(agg2, scaled2, dis, b2r, npad, d, d2)
    return (log_probs[:n], logits[:n])


# speedup vs baseline: 1.0036x; 1.0036x over previous
import functools

import jax
import jax.numpy as jnp
from jax import lax
from jax.experimental import pallas as pl
from jax.experimental.pallas import tpu as pltpu
from jax.experimental.pallas import tpu_sc as plsc

NC = 2
NS = 16
NW = NC * NS
LANE = 128
RING = 2
HALVES = 2


def _tc_mm1(x_pad, W1, npad, d):
    grid = npad // 256

    def body(x_ref, w_ref, o_ref):
        o_ref[...] = jnp.dot(x_ref[...], w_ref[...],
                             preferred_element_type=jnp.float32)

    return pl.pallas_call(
        body,
        grid=(grid,),
        in_specs=[
            pl.BlockSpec((256, d), lambda r: (r, 0)),
            pl.BlockSpec((d, d), lambda r: (0, 0)),
        ],
        out_specs=pl.BlockSpec((256, d), lambda r: (r, 0)),
        out_shape=jax.ShapeDtypeStruct((npad, d), jnp.float32),
    )(x_pad, W1)


def _sc_hist(dst3, npad, chunks):
    rows_per_tile = npad // NS
    zc = rows_per_tile // LANE
    mesh = plsc.VectorSubcoreMesh(core_axis_name="c", subcore_axis_name="s",
                                  num_cores=NC, num_subcores=NS)

    @functools.partial(
        pl.kernel,
        out_type=jax.ShapeDtypeStruct((NC, npad, 16), jnp.float32),
        mesh=mesh,
        scratch_types=[
            pltpu.VMEM((chunks, LANE), jnp.int32),
            pltpu.VMEM((LANE, 16), jnp.float32),
            pltpu.VMEM((LANE, 16), jnp.float32),
            pltpu.VMEM_SHARED((npad, 16), jnp.float32),
        ],
    )
    def hist(dst_hbm, out, dst_v, zb_v, ones_v, shared):
        c = lax.axis_index("c")
        s = lax.axis_index("s")
        wid = c * NS + s
        pltpu.sync_copy(dst_hbm.at[wid], dst_v)

        def init_body(i, _):
            zb_v[i, :] = jnp.zeros((16,), jnp.float32)
            ones_v[i, :] = jnp.ones((16,), jnp.float32)
            return _
        lax.fori_loop(0, LANE, init_body, None)
        for k in range(zc):
            pltpu.sync_copy(
                zb_v, shared.at[pl.ds(s * rows_per_tile + k * LANE, LANE)])
        plsc.subcore_barrier()

        def body(j, _):
            pltpu.sync_copy(ones_v, shared.at[dst_v.at[j]], add=True)
            return _
        lax.fori_loop(0, chunks, body, None)
        plsc.subcore_barrier()
        pltpu.sync_copy(shared.at[pl.ds(s * rows_per_tile, rows_per_tile)],
                        out.at[c, pl.ds(s * rows_per_tile, rows_per_tile)])

    return hist(dst3)


def _sc_agg(table, src3, dst3, npad, chunks, d):
    rows_per_tile = npad // NS
    zc = rows_per_tile // LANE
    mesh = plsc.VectorSubcoreMesh(core_axis_name="c", subcore_axis_name="s",
                                  num_cores=NC, num_subcores=NS)

    hchunks = chunks // HALVES

    @functools.partial(
        pl.kernel,
        out_type=jax.ShapeDtypeStruct((NC, npad, d), jnp.float32),
        mesh=mesh,
        scratch_types=[
            pltpu.VMEM((hchunks, LANE), jnp.int32),
            pltpu.VMEM((hchunks, LANE), jnp.int32),
            pltpu.VMEM((RING, LANE, d), jnp.float32),
            pltpu.VMEM_SHARED((npad, d), jnp.float32),
            pltpu.SemaphoreType.DMA,
            pltpu.SemaphoreType.DMA,
            pltpu.SemaphoreType.DMA,
        ],
    )
    def agg(table_hbm, src_hbm, dst_hbm, out, src_v, dst_v, rows_v, shared,
            gsem0, gsem1, ssem):
        c = lax.axis_index("c")
        s = lax.axis_index("s")
        wid = c * NS + s

        def zero_body(i, _):
            for j in range(d // 16):
                rows_v[0, i, pl.ds(j * 16, 16)] = jnp.zeros((16,), jnp.float32)
            return _
        lax.fori_loop(0, LANE, zero_body, None)
        for k in range(zc):
            pltpu.sync_copy(rows_v.at[0],
                            shared.at[pl.ds(s * rows_per_tile + k * LANE,
                                            LANE)])
        plsc.subcore_barrier()

        def half(h, _):
            pltpu.sync_copy(src_hbm.at[wid, pl.ds(h * hchunks, hchunks)],
                            src_v)
            pltpu.sync_copy(dst_hbm.at[wid, pl.ds(h * hchunks, hchunks)],
                            dst_v)

            gsems = [gsem0, gsem1]

            def group(g, _):
                j = g * RING
                gds = [
                    pltpu.async_copy(table_hbm.at[src_v.at[j + b]],
                                     rows_v.at[b], gsems[b])
                    for b in range(RING)
                ]
                sds = []
                for b in range(RING):
                    gds[b].wait()
                    sds.append(
                        pltpu.async_copy(rows_v.at[b],
                                         shared.at[dst_v.at[j + b]], ssem,
                                         add=True))
                for dsc in sds:
                    dsc.wait()
                return _
            lax.fori_loop(0, hchunks // RING, group, None)
            return _
        lax.fori_loop(0, HALVES, half, None)
        plsc.subcore_barrier()
        pltpu.sync_copy(shared.at[pl.ds(s * rows_per_tile, rows_per_tile)],
                        out.at[c, pl.ds(s * rows_per_tile, rows_per_tile)])

    return agg(table, src3, dst3)


def _tc_scale1(h1, histp, npad, d):
    grid = npad // 256

    def body(h_ref, hp_ref, s_ref, dis_ref):
        deg = 1.0 + hp_ref[0] + hp_ref[1]
        dis = lax.rsqrt(deg)
        dis_ref[...] = dis
        s_ref[...] = h_ref[...] * dis[:, 0:1]

    return pl.pallas_call(
        body,
        grid=(grid,),
        in_specs=[
            pl.BlockSpec((256, d), lambda r: (r, 0)),
            pl.BlockSpec((NC, 256, 16), lambda r: (0, r, 0)),
        ],
        out_specs=[
            pl.BlockSpec((256, d), lambda r: (r, 0)),
            pl.BlockSpec((256, 16), lambda r: (r, 0)),
        ],
        out_shape=[
            jax.ShapeDtypeStruct((npad, d), jnp.float32),
            jax.ShapeDtypeStruct((npad, 16), jnp.float32),
        ],
    )(h1, histp)


def _tc_l2(agg1, scaled1, dis, b1, W2, npad, d, d2):
    grid = npad // 256

    def body(a_ref, s1_ref, dis_ref, b1_ref, w2_ref, o_ref):
        di = dis_ref[:, 0:1]
        u = di * (a_ref[0] + a_ref[1] + s1_ref[...]) + b1_ref[...]
        h = jnp.maximum(u, 0.0)
        s2 = di * jnp.dot(h, w2_ref[...], preferred_element_type=jnp.float32)
        o_ref[...] = jnp.concatenate(
            [s2, jnp.zeros((s2.shape[0], d - d2), jnp.float32)], axis=1)

    return pl.pallas_call(
        body,
        grid=(grid,),
        in_specs=[
            pl.BlockSpec((NC, 256, d), lambda r: (0, r, 0)),
            pl.BlockSpec((256, d), lambda r: (r, 0)),
            pl.BlockSpec((256, 16), lambda r: (r, 0)),
            pl.BlockSpec((1, d), lambda r: (0, 0)),
            pl.BlockSpec((d, d2), lambda r: (0, 0)),
        ],
        out_specs=pl.BlockSpec((256, d), lambda r: (r, 0)),
        out_shape=jax.ShapeDtypeStruct((npad, d), jnp.float32),
    )(agg1, scaled1, dis, b1, W2)


def _tc_out(agg2, scaled2, dis, b2, npad, d, d2):
    grid = npad // 256

    def body(a_ref, s2_ref, dis_ref, b2_ref, lp_ref, lg_ref):
        di = dis_ref[:, 0:1]
        full = di * (a_ref[0] + a_ref[1] + s2_ref[...])
        logits = full[:, :d2] + b2_ref[...]
        lg_ref[...] = logits
        m = jnp.max(logits, axis=1, keepdims=True)
        e = jnp.exp(logits - m)
        lse = m + jnp.log(jnp.sum(e, axis=1, keepdims=True))
        lp_ref[...] = logits - lse

    return pl.pallas_call(
        body,
        grid=(grid,),
        in_specs=[
            pl.BlockSpec((NC, 256, d), lambda r: (0, r, 0)),
            pl.BlockSpec((256, d), lambda r: (r, 0)),
            pl.BlockSpec((256, 16), lambda r: (r, 0)),
            pl.BlockSpec((1, d2), lambda r: (0, 0)),
        ],
        out_specs=[
            pl.BlockSpec((256, d2), lambda r: (r, 0)),
            pl.BlockSpec((256, d2), lambda r: (r, 0)),
        ],
        out_shape=[
            jax.ShapeDtypeStruct((npad, d2), jnp.float32),
            jax.ShapeDtypeStruct((npad, d2), jnp.float32),
        ],
    )(agg2, scaled2, dis, b2)


def kernel(x, edge_index, W1, b1, W2, b2):
    n, d = x.shape
    d2 = W2.shape[1]
    e = edge_index.shape[1]

    npad = -(-n // 2048) * 2048
    chunks = -(-e // (NW * LANE))
    gran = RING * HALVES
    chunks = -(-chunks // gran) * gran
    epad = NW * chunks * LANE

    x_pad = jnp.pad(x, ((0, npad - n), (0, 0)))
    pad_idx = jnp.full((epad - e,), npad - 1, jnp.int32)
    src3 = jnp.concatenate([edge_index[0].astype(jnp.int32), pad_idx]
                           ).reshape(NW, chunks, LANE)
    dst3 = jnp.concatenate([edge_index[1].astype(jnp.int32), pad_idx]
                           ).reshape(NW, chunks, LANE)
    b1r = b1.reshape(1, d)
    b2r = b2.reshape(1, d2)

    h1 = _tc_mm1(x_pad, W1, npad, d)
    histp = _sc_hist(dst3, npad, chunks)
    scaled1, dis = _tc_scale1(h1, histp, npad, d)
    agg1 = _sc_agg(scaled1, src3, dst3, npad, chunks, d)
    scaled2 = _tc_l2(agg1, scaled1, dis, b1r, W2, npad, d, d2)
    agg2 = _sc_agg(scaled2, src3, dst3, npad, chunks, d)
    log_probs, logits = _tc_out(agg2, scaled2, dis, b2r, npad, d, d2)
    return (log_probs[:n], logits[:n])


# speedup vs baseline: 2.2178x; 2.2099x over previous
import functools

import jax
import jax.numpy as jnp
from jax import lax
from jax.experimental import pallas as pl
from jax.experimental.pallas import tpu as pltpu
from jax.experimental.pallas import tpu_sc as plsc

NC = 2
NS = 16
NW = NC * NS
LANE = 128
RING = 2
HALVES = 2


def _tc_mm1(x_pad, W1, npad, d):
    grid = npad // 256

    def body(x_ref, w_ref, o_ref):
        o_ref[...] = jnp.dot(x_ref[...], w_ref[...],
                             preferred_element_type=jnp.float32)

    return pl.pallas_call(
        body,
        grid=(grid,),
        in_specs=[
            pl.BlockSpec((256, d), lambda r: (r, 0)),
            pl.BlockSpec((d, d), lambda r: (0, 0)),
        ],
        out_specs=pl.BlockSpec((256, d), lambda r: (r, 0)),
        out_shape=jax.ShapeDtypeStruct((npad, d), jnp.float32),
    )(x_pad, W1)


def _sc_hist(dst3, npad, chunks):
    rows_per_tile = npad // NS
    zc = rows_per_tile // LANE
    mesh = plsc.VectorSubcoreMesh(core_axis_name="c", subcore_axis_name="s",
                                  num_cores=NC, num_subcores=NS)

    @functools.partial(
        pl.kernel,
        out_type=jax.ShapeDtypeStruct((NC, npad, 16), jnp.float32),
        mesh=mesh,
        scratch_types=[
            pltpu.VMEM((chunks, LANE), jnp.int32),
            pltpu.VMEM((LANE, 16), jnp.float32),
            pltpu.VMEM((LANE, 16), jnp.float32),
            pltpu.VMEM_SHARED((npad, 16), jnp.float32),
        ],
    )
    def hist(dst_hbm, out, dst_v, zb_v, ones_v, shared):
        c = lax.axis_index("c")
        s = lax.axis_index("s")
        wid = c * NS + s
        pltpu.sync_copy(dst_hbm.at[wid], dst_v)

        def init_body(i, _):
            zb_v[i, :] = jnp.zeros((16,), jnp.float32)
            ones_v[i, :] = jnp.ones((16,), jnp.float32)
            return _
        lax.fori_loop(0, LANE, init_body, None)
        for k in range(zc):
            pltpu.sync_copy(
                zb_v, shared.at[pl.ds(s * rows_per_tile + k * LANE, LANE)])
        plsc.subcore_barrier()

        def body(j, _):
            pltpu.sync_copy(ones_v, shared.at[dst_v.at[j]], add=True)
            return _
        lax.fori_loop(0, chunks, body, None)
        plsc.subcore_barrier()
        pltpu.sync_copy(shared.at[pl.ds(s * rows_per_tile, rows_per_tile)],
                        out.at[c, pl.ds(s * rows_per_tile, rows_per_tile)])

    return hist(dst3)


def _sc_agg(table, src3, dst3, npad, chunks, d):
    rows_per_tile = npad // NS
    zc = rows_per_tile // LANE
    mesh = plsc.VectorSubcoreMesh(core_axis_name="c", subcore_axis_name="s",
                                  num_cores=NC, num_subcores=NS)

    @functools.partial(
        pl.kernel,
        out_type=jax.ShapeDtypeStruct((NC, npad, d), jnp.float32),
        mesh=mesh,
        scratch_types=[
            pltpu.VMEM((chunks, LANE), jnp.int32),
            pltpu.VMEM((chunks, LANE), jnp.int32),
            pltpu.VMEM((LANE, d), jnp.float32),
            pltpu.VMEM_SHARED((npad, d), jnp.float32),
            pltpu.SemaphoreType.DMA,
        ],
    )
    def agg(table_hbm, src_hbm, dst_hbm, out, src_v, dst_v, rows_v, shared,
            sem):
        c = lax.axis_index("c")
        s = lax.axis_index("s")
        wid = c * NS + s
        pltpu.sync_copy(src_hbm.at[wid], src_v)
        pltpu.sync_copy(dst_hbm.at[wid], dst_v)

        def zero_body(i, _):
            for j in range(d // 16):
                rows_v[i, pl.ds(j * 16, 16)] = jnp.zeros((16,), jnp.float32)
            return _
        lax.fori_loop(0, LANE, zero_body, None)
        for k in range(zc):
            pltpu.sync_copy(rows_v,
                            shared.at[pl.ds(s * rows_per_tile + k * LANE,
                                            LANE)])
        plsc.subcore_barrier()

        def body(j, _):
            pltpu.async_copy(table_hbm.at[src_v.at[j]], rows_v, sem).wait()
            pltpu.sync_copy(rows_v, shared.at[dst_v.at[j]], add=True)
            return _
        lax.fori_loop(0, chunks, body, None)
        plsc.subcore_barrier()
        pltpu.sync_copy(shared.at[pl.ds(s * rows_per_tile, rows_per_tile)],
                        out.at[c, pl.ds(s * rows_per_tile, rows_per_tile)])

    return agg(table, src3, dst3)


def _tc_scale1(h1, histp, npad, d):
    grid = npad // 256

    def body(h_ref, hp_ref, s_ref, dis_ref):
        deg = 1.0 + hp_ref[0] + hp_ref[1]
        dis = lax.rsqrt(deg)
        dis_ref[...] = dis
        s_ref[...] = h_ref[...] * dis[:, 0:1]

    return pl.pallas_call(
        body,
        grid=(grid,),
        in_specs=[
            pl.BlockSpec((256, d), lambda r: (r, 0)),
            pl.BlockSpec((NC, 256, 16), lambda r: (0, r, 0)),
        ],
        out_specs=[
            pl.BlockSpec((256, d), lambda r: (r, 0)),
            pl.BlockSpec((256, 16), lambda r: (r, 0)),
        ],
        out_shape=[
            jax.ShapeDtypeStruct((npad, d), jnp.float32),
            jax.ShapeDtypeStruct((npad, 16), jnp.float32),
        ],
    )(h1, histp)


def _tc_l2(agg1, scaled1, dis, b1, W2, npad, d, d2):
    grid = npad // 256

    def body(a_ref, s1_ref, dis_ref, b1_ref, w2_ref, o_ref):
        di = dis_ref[:, 0:1]
        u = di * (a_ref[0] + a_ref[1] + s1_ref[...]) + b1_ref[...]
        h = jnp.maximum(u, 0.0)
        s2 = di * jnp.dot(h, w2_ref[...], preferred_element_type=jnp.float32)
        o_ref[...] = jnp.concatenate(
            [s2, jnp.zeros((s2.shape[0], d - d2), jnp.float32)], axis=1)

    return pl.pallas_call(
        body,
        grid=(grid,),
        in_specs=[
            pl.BlockSpec((NC, 256, d), lambda r: (0, r, 0)),
            pl.BlockSpec((256, d), lambda r: (r, 0)),
            pl.BlockSpec((256, 16), lambda r: (r, 0)),
            pl.BlockSpec((1, d), lambda r: (0, 0)),
            pl.BlockSpec((d, d2), lambda r: (0, 0)),
        ],
        out_specs=pl.BlockSpec((256, d), lambda r: (r, 0)),
        out_shape=jax.ShapeDtypeStruct((npad, d), jnp.float32),
    )(agg1, scaled1, dis, b1, W2)


def _tc_out(agg2, scaled2, dis, b2, npad, d, d2):
    grid = npad // 256

    def body(a_ref, s2_ref, dis_ref, b2_ref, lp_ref, lg_ref):
        di = dis_ref[:, 0:1]
        full = di * (a_ref[0] + a_ref[1] + s2_ref[...])
        logits = full[:, :d2] + b2_ref[...]
        lg_ref[...] = logits
        m = jnp.max(logits, axis=1, keepdims=True)
        e = jnp.exp(logits - m)
        lse = m + jnp.log(jnp.sum(e, axis=1, keepdims=True))
        lp_ref[...] = logits - lse

    return pl.pallas_call(
        body,
        grid=(grid,),
        in_specs=[
            pl.BlockSpec((NC, 256, d), lambda r: (0, r, 0)),
            pl.BlockSpec((256, d), lambda r: (r, 0)),
            pl.BlockSpec((256, 16), lambda r: (r, 0)),
            pl.BlockSpec((1, d2), lambda r: (0, 0)),
        ],
        out_specs=[
            pl.BlockSpec((256, d2), lambda r: (r, 0)),
            pl.BlockSpec((256, d2), lambda r: (r, 0)),
        ],
        out_shape=[
            jax.ShapeDtypeStruct((npad, d2), jnp.float32),
            jax.ShapeDtypeStruct((npad, d2), jnp.float32),
        ],
    )(agg2, scaled2, dis, b2)


def kernel(x, edge_index, W1, b1, W2, b2):
    n, d = x.shape
    d2 = W2.shape[1]
    e = edge_index.shape[1]

    npad = -(-n // 2048) * 2048
    if npad == n:
        npad += 2048
    chunks = -(-e // (NW * LANE))
    epad = NW * chunks * LANE

    x_pad = jnp.pad(x, ((0, npad - n), (0, 0)))
    pad_idx = n + (jnp.arange(epad - e, dtype=jnp.int32) % (npad - n))
    src3 = jnp.concatenate([edge_index[0].astype(jnp.int32), pad_idx]
                           ).reshape(NW, chunks, LANE)
    dst3 = jnp.concatenate([edge_index[1].astype(jnp.int32), pad_idx]
                           ).reshape(NW, chunks, LANE)
    b1r = b1.reshape(1, d)
    b2r = b2.reshape(1, d2)

    h1 = _tc_mm1(x_pad, W1, npad, d)
    histp = _sc_hist(dst3, npad, chunks)
    scaled1, dis = _tc_scale1(h1, histp, npad, d)
    agg1 = _sc_agg(scaled1, src3, dst3, npad, chunks, d)
    scaled2 = _tc_l2(agg1, scaled1, dis, b1r, W2, npad, d, d2)
    agg2 = _sc_agg(scaled2, src3, dst3, npad, chunks, d)
    log_probs, logits = _tc_out(agg2, scaled2, dis, b2r, npad, d, d2)
    return (log_probs[:n], logits[:n])


# speedup vs baseline: 2.4488x; 1.1042x over previous
import functools

import jax
import jax.numpy as jnp
from jax import lax
from jax.experimental import pallas as pl
from jax.experimental.pallas import tpu as pltpu
from jax.experimental.pallas import tpu_sc as plsc

NC = 2
NS = 16
NW = NC * NS
LANE = 128
RING = 2
HALVES = 2


def _tc_mm1(x_pad, W1, npad, d):
    grid = npad // 256

    def body(x_ref, w_ref, o_ref):
        o_ref[...] = jnp.dot(x_ref[...], w_ref[...],
                             preferred_element_type=jnp.float32)

    return pl.pallas_call(
        body,
        grid=(grid,),
        in_specs=[
            pl.BlockSpec((256, d), lambda r: (r, 0)),
            pl.BlockSpec((d, d), lambda r: (0, 0)),
        ],
        out_specs=pl.BlockSpec((256, d), lambda r: (r, 0)),
        out_shape=jax.ShapeDtypeStruct((npad, d), jnp.float32),
    )(x_pad, W1)


def _sc_hist(dst3, npad, chunks):
    rows_per_tile = npad // NS
    zc = rows_per_tile // LANE
    mesh = plsc.VectorSubcoreMesh(core_axis_name="c", subcore_axis_name="s",
                                  num_cores=NC, num_subcores=NS)

    @functools.partial(
        pl.kernel,
        out_type=jax.ShapeDtypeStruct((NC, npad, 16), jnp.float32),
        mesh=mesh,
        scratch_types=[
            pltpu.VMEM((chunks, LANE), jnp.int32),
            pltpu.VMEM((LANE, 16), jnp.float32),
            pltpu.VMEM((LANE, 16), jnp.float32),
            pltpu.VMEM_SHARED((npad, 16), jnp.float32),
        ],
    )
    def hist(dst_hbm, out, dst_v, zb_v, ones_v, shared):
        c = lax.axis_index("c")
        s = lax.axis_index("s")
        wid = c * NS + s
        pltpu.sync_copy(dst_hbm.at[wid], dst_v)

        def init_body(i, _):
            zb_v[i, :] = jnp.zeros((16,), jnp.float32)
            ones_v[i, :] = jnp.ones((16,), jnp.float32)
            return _
        lax.fori_loop(0, LANE, init_body, None)
        for k in range(zc):
            pltpu.sync_copy(
                zb_v, shared.at[pl.ds(s * rows_per_tile + k * LANE, LANE)])
        plsc.subcore_barrier()

        def body(j, _):
            pltpu.sync_copy(ones_v, shared.at[dst_v.at[j]], add=True)
            return _
        lax.fori_loop(0, chunks, body, None)
        plsc.subcore_barrier()
        pltpu.sync_copy(shared.at[pl.ds(s * rows_per_tile, rows_per_tile)],
                        out.at[c, pl.ds(s * rows_per_tile, rows_per_tile)])

    return hist(dst3)


def _sc_agg(table, src3, dst3, npad, chunks, d):
    rows_per_tile = npad // NS
    zc = rows_per_tile // LANE
    mesh = plsc.VectorSubcoreMesh(core_axis_name="c", subcore_axis_name="s",
                                  num_cores=NC, num_subcores=NS)

    hchunks = chunks // HALVES

    @functools.partial(
        pl.kernel,
        out_type=jax.ShapeDtypeStruct((NC, npad, d), jnp.float32),
        mesh=mesh,
        scratch_types=[
            pltpu.VMEM((hchunks, LANE), jnp.int32),
            pltpu.VMEM((hchunks, LANE), jnp.int32),
            pltpu.VMEM((RING, LANE, d), jnp.float32),
            pltpu.VMEM_SHARED((npad, d), jnp.float32),
            pltpu.SemaphoreType.DMA,
            pltpu.SemaphoreType.DMA,
        ],
    )
    def agg(table_hbm, src_hbm, dst_hbm, out, src_v, dst_v, rows_v, shared,
            gsem0, gsem1):
        c = lax.axis_index("c")
        s = lax.axis_index("s")
        wid = c * NS + s

        def zero_body(i, _):
            for j in range(d // 16):
                rows_v[0, i, pl.ds(j * 16, 16)] = jnp.zeros((16,), jnp.float32)
            return _
        lax.fori_loop(0, LANE, zero_body, None)
        for k in range(zc):
            pltpu.sync_copy(rows_v.at[0],
                            shared.at[pl.ds(s * rows_per_tile + k * LANE,
                                            LANE)])
        plsc.subcore_barrier()

        gsems = [gsem0, gsem1]

        def half(h, _):
            pltpu.sync_copy(src_hbm.at[wid, pl.ds(h * hchunks, hchunks)],
                            src_v)
            pltpu.sync_copy(dst_hbm.at[wid, pl.ds(h * hchunks, hchunks)],
                            dst_v)

            def group(g, _):
                j = g * RING
                gds = [
                    pltpu.async_copy(table_hbm.at[src_v.at[j + b]],
                                     rows_v.at[b], gsems[b])
                    for b in range(RING)
                ]
                for b in range(RING):
                    gds[b].wait()
                    pltpu.sync_copy(rows_v.at[b], shared.at[dst_v.at[j + b]],
                                    add=True)
                return _
            lax.fori_loop(0, hchunks // RING, group, None)
            return _
        lax.fori_loop(0, HALVES, half, None)
        plsc.subcore_barrier()
        pltpu.sync_copy(shared.at[pl.ds(s * rows_per_tile, rows_per_tile)],
                        out.at[c, pl.ds(s * rows_per_tile, rows_per_tile)])

    return agg(table, src3, dst3)


def _tc_scale1(h1, histp, npad, d):
    grid = npad // 256

    def body(h_ref, hp_ref, s_ref, dis_ref):
        deg = 1.0 + hp_ref[0] + hp_ref[1]
        dis = lax.rsqrt(deg)
        dis_ref[...] = dis
        s_ref[...] = h_ref[...] * dis[:, 0:1]

    return pl.pallas_call(
        body,
        grid=(grid,),
        in_specs=[
            pl.BlockSpec((256, d), lambda r: (r, 0)),
            pl.BlockSpec((NC, 256, 16), lambda r: (0, r, 0)),
        ],
        out_specs=[
            pl.BlockSpec((256, d), lambda r: (r, 0)),
            pl.BlockSpec((256, 16), lambda r: (r, 0)),
        ],
        out_shape=[
            jax.ShapeDtypeStruct((npad, d), jnp.float32),
            jax.ShapeDtypeStruct((npad, 16), jnp.float32),
        ],
    )(h1, histp)


def _tc_l2(agg1, scaled1, dis, b1, W2, npad, d, d2):
    grid = npad // 256

    def body(a_ref, s1_ref, dis_ref, b1_ref, w2_ref, o_ref):
        di = dis_ref[:, 0:1]
        u = di * (a_ref[0] + a_ref[1] + s1_ref[...]) + b1_ref[...]
        h = jnp.maximum(u, 0.0)
        s2 = di * jnp.dot(h, w2_ref[...], preferred_element_type=jnp.float32)
        o_ref[...] = jnp.concatenate(
            [s2, jnp.zeros((s2.shape[0], d - d2), jnp.float32)], axis=1)

    return pl.pallas_call(
        body,
        grid=(grid,),
        in_specs=[
            pl.BlockSpec((NC, 256, d), lambda r: (0, r, 0)),
            pl.BlockSpec((256, d), lambda r: (r, 0)),
            pl.BlockSpec((256, 16), lambda r: (r, 0)),
            pl.BlockSpec((1, d), lambda r: (0, 0)),
            pl.BlockSpec((d, d2), lambda r: (0, 0)),
        ],
        out_specs=pl.BlockSpec((256, d), lambda r: (r, 0)),
        out_shape=jax.ShapeDtypeStruct((npad, d), jnp.float32),
    )(agg1, scaled1, dis, b1, W2)


def _tc_out(agg2, scaled2, dis, b2, npad, d, d2):
    grid = npad // 256

    def body(a_ref, s2_ref, dis_ref, b2_ref, lp_ref, lg_ref):
        di = dis_ref[:, 0:1]
        full = di * (a_ref[0] + a_ref[1] + s2_ref[...])
        logits = full[:, :d2] + b2_ref[...]
        lg_ref[...] = logits
        m = jnp.max(logits, axis=1, keepdims=True)
        e = jnp.exp(logits - m)
        lse = m + jnp.log(jnp.sum(e, axis=1, keepdims=True))
        lp_ref[...] = logits - lse

    return pl.pallas_call(
        body,
        grid=(grid,),
        in_specs=[
            pl.BlockSpec((NC, 256, d), lambda r: (0, r, 0)),
            pl.BlockSpec((256, d), lambda r: (r, 0)),
            pl.BlockSpec((256, 16), lambda r: (r, 0)),
            pl.BlockSpec((1, d2), lambda r: (0, 0)),
        ],
        out_specs=[
            pl.BlockSpec((256, d2), lambda r: (r, 0)),
            pl.BlockSpec((256, d2), lambda r: (r, 0)),
        ],
        out_shape=[
            jax.ShapeDtypeStruct((npad, d2), jnp.float32),
            jax.ShapeDtypeStruct((npad, d2), jnp.float32),
        ],
    )(agg2, scaled2, dis, b2)


def kernel(x, edge_index, W1, b1, W2, b2):
    n, d = x.shape
    d2 = W2.shape[1]
    e = edge_index.shape[1]

    npad = -(-n // 2048) * 2048
    if npad == n:
        npad += 2048
    chunks = -(-e // (NW * LANE))
    gran = RING * HALVES
    chunks = -(-chunks // gran) * gran
    epad = NW * chunks * LANE

    x_pad = jnp.pad(x, ((0, npad - n), (0, 0)))
    pad_idx = n + (jnp.arange(epad - e, dtype=jnp.int32) % (npad - n))
    src3 = jnp.concatenate([edge_index[0].astype(jnp.int32), pad_idx]
                           ).reshape(NW, chunks, LANE)
    dst3 = jnp.concatenate([edge_index[1].astype(jnp.int32), pad_idx]
                           ).reshape(NW, chunks, LANE)
    b1r = b1.reshape(1, d)
    b2r = b2.reshape(1, d2)

    h1 = _tc_mm1(x_pad, W1, npad, d)
    histp = _sc_hist(dst3, npad, chunks)
    scaled1, dis = _tc_scale1(h1, histp, npad, d)
    agg1 = _sc_agg(scaled1, src3, dst3, npad, chunks, d)
    scaled2 = _tc_l2(agg1, scaled1, dis, b1r, W2, npad, d, d2)
    agg2 = _sc_agg(scaled2, src3, dst3, npad, chunks, d)
    log_probs, logits = _tc_out(agg2, scaled2, dis, b2r, npad, d, d2)
    return (log_probs[:n], logits[:n])


# speedup vs baseline: 2.4761x; 1.0111x over previous
import functools

import jax
import jax.numpy as jnp
from jax import lax
from jax.experimental import pallas as pl
from jax.experimental.pallas import tpu as pltpu
from jax.experimental.pallas import tpu_sc as plsc

NC = 2
NS = 16
NW = NC * NS
LANE = 128
RING = 2
HALVES = 2


def _tc_mm1(x_pad, W1, npad, d):
    grid = npad // 256

    def body(x_ref, w_ref, o_ref):
        o_ref[...] = jnp.dot(x_ref[...], w_ref[...],
                             preferred_element_type=jnp.float32)

    return pl.pallas_call(
        body,
        grid=(grid,),
        in_specs=[
            pl.BlockSpec((256, d), lambda r: (r, 0)),
            pl.BlockSpec((d, d), lambda r: (0, 0)),
        ],
        out_specs=pl.BlockSpec((256, d), lambda r: (r, 0)),
        out_shape=jax.ShapeDtypeStruct((npad, d), jnp.float32),
    )(x_pad, W1)


def _sc_hist(dst3, npad, chunks):
    rows_per_tile = npad // NS
    zc = rows_per_tile // LANE
    mesh = plsc.VectorSubcoreMesh(core_axis_name="c", subcore_axis_name="s",
                                  num_cores=NC, num_subcores=NS)

    @functools.partial(
        pl.kernel,
        out_type=jax.ShapeDtypeStruct((NC, npad, 16), jnp.float32),
        mesh=mesh,
        scratch_types=[
            pltpu.VMEM((chunks, LANE), jnp.int32),
            pltpu.VMEM((LANE, 16), jnp.float32),
            pltpu.VMEM((LANE, 16), jnp.float32),
            pltpu.VMEM_SHARED((npad, 16), jnp.float32),
        ],
    )
    def hist(dst_hbm, out, dst_v, zb_v, ones_v, shared):
        c = lax.axis_index("c")
        s = lax.axis_index("s")
        wid = c * NS + s
        pltpu.sync_copy(dst_hbm.at[wid], dst_v)

        def init_body(i, _):
            zb_v[i, :] = jnp.zeros((16,), jnp.float32)
            ones_v[i, :] = jnp.ones((16,), jnp.float32)
            return _
        lax.fori_loop(0, LANE, init_body, None)
        for k in range(zc):
            pltpu.sync_copy(
                zb_v, shared.at[pl.ds(s * rows_per_tile + k * LANE, LANE)])
        plsc.subcore_barrier()

        def body(j, _):
            pltpu.sync_copy(ones_v, shared.at[dst_v.at[j]], add=True)
            return _
        lax.fori_loop(0, chunks, body, None)
        plsc.subcore_barrier()
        pltpu.sync_copy(shared.at[pl.ds(s * rows_per_tile, rows_per_tile)],
                        out.at[c, pl.ds(s * rows_per_tile, rows_per_tile)])

    return hist(dst3)


def _sc_agg(table, src3, dst3, npad, chunks, d):
    rows_per_tile = npad // NS
    zc = rows_per_tile // LANE
    mesh = plsc.VectorSubcoreMesh(core_axis_name="c", subcore_axis_name="s",
                                  num_cores=NC, num_subcores=NS)

    hchunks = chunks // HALVES

    @functools.partial(
        pl.kernel,
        out_type=jax.ShapeDtypeStruct((NC, npad, d), jnp.float32),
        mesh=mesh,
        scratch_types=[
            pltpu.VMEM((hchunks, LANE), jnp.int32),
            pltpu.VMEM((hchunks, LANE), jnp.int32),
            pltpu.VMEM((RING, LANE, d), jnp.float32),
            pltpu.VMEM_SHARED((npad, d), jnp.float32),
            pltpu.SemaphoreType.DMA,
            pltpu.SemaphoreType.DMA,
            pltpu.SemaphoreType.DMA,
        ],
    )
    def agg(table_hbm, src_hbm, dst_hbm, out, src_v, dst_v, rows_v, shared,
            gsem0, gsem1, ssem):
        c = lax.axis_index("c")
        s = lax.axis_index("s")
        wid = c * NS + s

        def zero_body(i, _):
            for j in range(d // 16):
                rows_v[0, i, pl.ds(j * 16, 16)] = jnp.zeros((16,), jnp.float32)
            return _
        lax.fori_loop(0, LANE, zero_body, None)
        for k in range(zc):
            pltpu.sync_copy(rows_v.at[0],
                            shared.at[pl.ds(s * rows_per_tile + k * LANE,
                                            LANE)])
        plsc.subcore_barrier()

        gsems = [gsem0, gsem1]

        def half(h, _):
            pltpu.sync_copy(src_hbm.at[wid, pl.ds(h * hchunks, hchunks)],
                            src_v)
            pltpu.sync_copy(dst_hbm.at[wid, pl.ds(h * hchunks, hchunks)],
                            dst_v)

            def group(g, _):
                j = g * RING
                gds = [
                    pltpu.async_copy(table_hbm.at[src_v.at[j + b]],
                                     rows_v.at[b], gsems[b])
                    for b in range(RING)
                ]
                sds = []
                for b in range(RING):
                    gds[b].wait()
                    sds.append(
                        pltpu.async_copy(rows_v.at[b],
                                         shared.at[dst_v.at[j + b]], ssem,
                                         add=True))
                for dsc in sds:
                    dsc.wait()
                return _
            lax.fori_loop(0, hchunks // RING, group, None)
            return _
        lax.fori_loop(0, HALVES, half, None)
        plsc.subcore_barrier()
        pltpu.sync_copy(shared.at[pl.ds(s * rows_per_tile, rows_per_tile)],
                        out.at[c, pl.ds(s * rows_per_tile, rows_per_tile)])

    return agg(table, src3, dst3)


def _tc_scale1(h1, histp, npad, d):
    grid = npad // 256

    def body(h_ref, hp_ref, s_ref, dis_ref):
        deg = 1.0 + hp_ref[0] + hp_ref[1]
        dis = lax.rsqrt(deg)
        dis_ref[...] = dis
        s_ref[...] = h_ref[...] * dis[:, 0:1]

    return pl.pallas_call(
        body,
        grid=(grid,),
        in_specs=[
            pl.BlockSpec((256, d), lambda r: (r, 0)),
            pl.BlockSpec((NC, 256, 16), lambda r: (0, r, 0)),
        ],
        out_specs=[
            pl.BlockSpec((256, d), lambda r: (r, 0)),
            pl.BlockSpec((256, 16), lambda r: (r, 0)),
        ],
        out_shape=[
            jax.ShapeDtypeStruct((npad, d), jnp.float32),
            jax.ShapeDtypeStruct((npad, 16), jnp.float32),
        ],
    )(h1, histp)


def _tc_l2(agg1, scaled1, dis, b1, W2, npad, d, d2):
    grid = npad // 256

    def body(a_ref, s1_ref, dis_ref, b1_ref, w2_ref, o_ref):
        di = dis_ref[:, 0:1]
        u = di * (a_ref[0] + a_ref[1] + s1_ref[...]) + b1_ref[...]
        h = jnp.maximum(u, 0.0)
        s2 = di * jnp.dot(h, w2_ref[...], preferred_element_type=jnp.float32)
        o_ref[...] = jnp.concatenate(
            [s2, jnp.zeros((s2.shape[0], d - d2), jnp.float32)], axis=1)

    return pl.pallas_call(
        body,
        grid=(grid,),
        in_specs=[
            pl.BlockSpec((NC, 256, d), lambda r: (0, r, 0)),
            pl.BlockSpec((256, d), lambda r: (r, 0)),
            pl.BlockSpec((256, 16), lambda r: (r, 0)),
            pl.BlockSpec((1, d), lambda r: (0, 0)),
            pl.BlockSpec((d, d2), lambda r: (0, 0)),
        ],
        out_specs=pl.BlockSpec((256, d), lambda r: (r, 0)),
        out_shape=jax.ShapeDtypeStruct((npad, d), jnp.float32),
    )(agg1, scaled1, dis, b1, W2)


def _tc_out(agg2, scaled2, dis, b2, npad, d, d2):
    grid = npad // 256

    def body(a_ref, s2_ref, dis_ref, b2_ref, lp_ref, lg_ref):
        di = dis_ref[:, 0:1]
        full = di * (a_ref[0] + a_ref[1] + s2_ref[...])
        logits = full[:, :d2] + b2_ref[...]
        lg_ref[...] = logits
        m = jnp.max(logits, axis=1, keepdims=True)
        e = jnp.exp(logits - m)
        lse = m + jnp.log(jnp.sum(e, axis=1, keepdims=True))
        lp_ref[...] = logits - lse

    return pl.pallas_call(
        body,
        grid=(grid,),
        in_specs=[
            pl.BlockSpec((NC, 256, d), lambda r: (0, r, 0)),
            pl.BlockSpec((256, d), lambda r: (r, 0)),
            pl.BlockSpec((256, 16), lambda r: (r, 0)),
            pl.BlockSpec((1, d2), lambda r: (0, 0)),
        ],
        out_specs=[
            pl.BlockSpec((256, d2), lambda r: (r, 0)),
            pl.BlockSpec((256, d2), lambda r: (r, 0)),
        ],
        out_shape=[
            jax.ShapeDtypeStruct((npad, d2), jnp.float32),
            jax.ShapeDtypeStruct((npad, d2), jnp.float32),
        ],
    )(agg2, scaled2, dis, b2)


def kernel(x, edge_index, W1, b1, W2, b2):
    n, d = x.shape
    d2 = W2.shape[1]
    e = edge_index.shape[1]

    npad = -(-n // 2048) * 2048
    if npad == n:
        npad += 2048
    chunks = -(-e // (NW * LANE))
    gran = RING * HALVES
    chunks = -(-chunks // gran) * gran
    epad = NW * chunks * LANE

    x_pad = jnp.pad(x, ((0, npad - n), (0, 0)))
    pad_idx = n + (jnp.arange(epad - e, dtype=jnp.int32) % (npad - n))
    src3 = jnp.concatenate([edge_index[0].astype(jnp.int32), pad_idx]
                           ).reshape(NW, chunks, LANE)
    dst3 = jnp.concatenate([edge_index[1].astype(jnp.int32), pad_idx]
                           ).reshape(NW, chunks, LANE)
    b1r = b1.reshape(1, d)
    b2r = b2.reshape(1, d2)

    h1 = _tc_mm1(x_pad, W1, npad, d)
    histp = _sc_hist(dst3, npad, chunks)
    scaled1, dis = _tc_scale1(h1, histp, npad, d)
    agg1 = _sc_agg(scaled1, src3, dst3, npad, chunks, d)
    scaled2 = _tc_l2(agg1, scaled1, dis, b1r, W2, npad, d, d2)
    agg2 = _sc_agg(scaled2, src3, dst3, npad, chunks, d)
    log_probs, logits = _tc_out(agg2, scaled2, dis, b2r, npad, d, d2)
    return (log_probs[:n], logits[:n])


# speedup vs baseline: 2.5301x; 1.0218x over previous
import functools

import jax
import jax.numpy as jnp
from jax import lax
from jax.experimental import pallas as pl
from jax.experimental.pallas import tpu as pltpu
from jax.experimental.pallas import tpu_sc as plsc

NC = 2
NS = 16
NW = NC * NS
LANE = 128
RING = 2
HALVES = 2


def _tc_scale1(x_pad, W1, histp, npad, d):
    grid = npad // 256

    def body(x_ref, w_ref, hp_ref, s_ref, dis_ref):
        h = jnp.dot(x_ref[...], w_ref[...], preferred_element_type=jnp.float32)
        deg = 1.0 + hp_ref[0] + hp_ref[1]
        dis = lax.rsqrt(deg)
        dis_ref[...] = dis
        s_ref[...] = h * dis[:, 0:1]

    return pl.pallas_call(
        body,
        grid=(grid,),
        in_specs=[
            pl.BlockSpec((256, d), lambda r: (r, 0)),
            pl.BlockSpec((d, d), lambda r: (0, 0)),
            pl.BlockSpec((NC, 256, 16), lambda r: (0, r, 0)),
        ],
        out_specs=[
            pl.BlockSpec((256, d), lambda r: (r, 0)),
            pl.BlockSpec((256, 16), lambda r: (r, 0)),
        ],
        out_shape=[
            jax.ShapeDtypeStruct((npad, d), jnp.float32),
            jax.ShapeDtypeStruct((npad, 16), jnp.float32),
        ],
    )(x_pad, W1, histp)


def _sc_hist(dst3, npad, chunks):
    rows_per_tile = npad // NS
    zc = rows_per_tile // LANE
    mesh = plsc.VectorSubcoreMesh(core_axis_name="c", subcore_axis_name="s",
                                  num_cores=NC, num_subcores=NS)

    @functools.partial(
        pl.kernel,
        out_type=jax.ShapeDtypeStruct((NC, npad, 16), jnp.float32),
        mesh=mesh,
        scratch_types=[
            pltpu.VMEM((chunks, LANE), jnp.int32),
            pltpu.VMEM((LANE, 16), jnp.float32),
            pltpu.VMEM((LANE, 16), jnp.float32),
            pltpu.VMEM_SHARED((npad, 16), jnp.float32),
        ],
    )
    def hist(dst_hbm, out, dst_v, zb_v, ones_v, shared):
        c = lax.axis_index("c")
        s = lax.axis_index("s")
        wid = c * NS + s
        pltpu.sync_copy(dst_hbm.at[wid], dst_v)

        def init_body(i, _):
            zb_v[i, :] = jnp.zeros((16,), jnp.float32)
            ones_v[i, :] = jnp.ones((16,), jnp.float32)
            return _
        lax.fori_loop(0, LANE, init_body, None)
        for k in range(zc):
            pltpu.sync_copy(
                zb_v, shared.at[pl.ds(s * rows_per_tile + k * LANE, LANE)])
        plsc.subcore_barrier()

        def body(j, _):
            pltpu.sync_copy(ones_v, shared.at[dst_v.at[j]], add=True)
            return _
        lax.fori_loop(0, chunks, body, None)
        plsc.subcore_barrier()
        pltpu.sync_copy(shared.at[pl.ds(s * rows_per_tile, rows_per_tile)],
                        out.at[c, pl.ds(s * rows_per_tile, rows_per_tile)])

    return hist(dst3)


def _sc_agg(table, src3, dst3, npad, chunks, d):
    rows_per_tile = npad // NS
    zc = rows_per_tile // LANE
    mesh = plsc.VectorSubcoreMesh(core_axis_name="c", subcore_axis_name="s",
                                  num_cores=NC, num_subcores=NS)

    hchunks = chunks // HALVES

    @functools.partial(
        pl.kernel,
        out_type=jax.ShapeDtypeStruct((NC, npad, d), jnp.float32),
        mesh=mesh,
        scratch_types=[
            pltpu.VMEM((hchunks, LANE), jnp.int32),
            pltpu.VMEM((hchunks, LANE), jnp.int32),
            pltpu.VMEM((RING, LANE, d), jnp.float32),
            pltpu.VMEM_SHARED((npad, d), jnp.float32),
            pltpu.SemaphoreType.DMA,
            pltpu.SemaphoreType.DMA,
        ],
    )
    def agg(table_hbm, src_hbm, dst_hbm, out, src_v, dst_v, rows_v, shared,
            gsem0, gsem1):
        c = lax.axis_index("c")
        s = lax.axis_index("s")
        wid = c * NS + s

        def zero_body(i, _):
            for j in range(d // 16):
                rows_v[0, i, pl.ds(j * 16, 16)] = jnp.zeros((16,), jnp.float32)
            return _
        lax.fori_loop(0, LANE, zero_body, None)
        for k in range(zc):
            pltpu.sync_copy(rows_v.at[0],
                            shared.at[pl.ds(s * rows_per_tile + k * LANE,
                                            LANE)])
        plsc.subcore_barrier()

        gsems = [gsem0, gsem1]

        def half(h, _):
            pltpu.sync_copy(src_hbm.at[wid, pl.ds(h * hchunks, hchunks)],
                            src_v)
            pltpu.sync_copy(dst_hbm.at[wid, pl.ds(h * hchunks, hchunks)],
                            dst_v)

            def group(g, _):
                j = g * RING
                gds = [
                    pltpu.async_copy(table_hbm.at[src_v.at[j + b]],
                                     rows_v.at[b], gsems[b])
                    for b in range(RING)
                ]
                for b in range(RING):
                    gds[b].wait()
                    pltpu.sync_copy(rows_v.at[b], shared.at[dst_v.at[j + b]],
                                    add=True)
                return _
            lax.fori_loop(0, hchunks // RING, group, None)
            return _
        lax.fori_loop(0, HALVES, half, None)
        plsc.subcore_barrier()
        pltpu.sync_copy(shared.at[pl.ds(s * rows_per_tile, rows_per_tile)],
                        out.at[c, pl.ds(s * rows_per_tile, rows_per_tile)])

    return agg(table, src3, dst3)


def _tc_l2(agg1, scaled1, dis, b1, W2, npad, d, d2):
    grid = npad // 256

    def body(a_ref, s1_ref, dis_ref, b1_ref, w2_ref, o_ref):
        di = dis_ref[:, 0:1]
        u = di * (a_ref[0] + a_ref[1] + s1_ref[...]) + b1_ref[...]
        h = jnp.maximum(u, 0.0)
        s2 = di * jnp.dot(h, w2_ref[...], preferred_element_type=jnp.float32)
        o_ref[...] = jnp.concatenate(
            [s2, jnp.zeros((s2.shape[0], d - d2), jnp.float32)], axis=1)

    return pl.pallas_call(
        body,
        grid=(grid,),
        in_specs=[
            pl.BlockSpec((NC, 256, d), lambda r: (0, r, 0)),
            pl.BlockSpec((256, d), lambda r: (r, 0)),
            pl.BlockSpec((256, 16), lambda r: (r, 0)),
            pl.BlockSpec((1, d), lambda r: (0, 0)),
            pl.BlockSpec((d, d2), lambda r: (0, 0)),
        ],
        out_specs=pl.BlockSpec((256, d), lambda r: (r, 0)),
        out_shape=jax.ShapeDtypeStruct((npad, d), jnp.float32),
    )(agg1, scaled1, dis, b1, W2)


def _tc_out(agg2, scaled2, dis, b2, n, d, d2):
    bn = next(b for b in (400, 256, 200, 128, 80, 16, 8) if n % b == 0)
    grid = n // bn

    def body(a_ref, s2_ref, dis_ref, b2_ref, lp_ref, lg_ref):
        di = dis_ref[:, 0:1]
        full = di * (a_ref[0] + a_ref[1] + s2_ref[...])
        logits = full[:, :d2] + b2_ref[...]
        lg_ref[...] = logits
        m = jnp.max(logits, axis=1, keepdims=True)
        e = jnp.exp(logits - m)
        lse = m + jnp.log(jnp.sum(e, axis=1, keepdims=True))
        lp_ref[...] = logits - lse

    return pl.pallas_call(
        body,
        grid=(grid,),
        in_specs=[
            pl.BlockSpec((NC, bn, d), lambda r: (0, r, 0)),
            pl.BlockSpec((bn, d), lambda r: (r, 0)),
            pl.BlockSpec((bn, 16), lambda r: (r, 0)),
            pl.BlockSpec((1, d2), lambda r: (0, 0)),
        ],
        out_specs=[
            pl.BlockSpec((bn, d2), lambda r: (r, 0)),
            pl.BlockSpec((bn, d2), lambda r: (r, 0)),
        ],
        out_shape=[
            jax.ShapeDtypeStruct((n, d2), jnp.float32),
            jax.ShapeDtypeStruct((n, d2), jnp.float32),
        ],
    )(agg2, scaled2, dis, b2)


def kernel(x, edge_index, W1, b1, W2, b2):
    n, d = x.shape
    d2 = W2.shape[1]
    e = edge_index.shape[1]

    npad = -(-n // 2048) * 2048
    if npad == n:
        npad += 2048
    chunks = -(-e // (NW * LANE))
    gran = RING * HALVES
    chunks = -(-chunks // gran) * gran
    epad = NW * chunks * LANE

    x_pad = jnp.pad(x, ((0, npad - n), (0, 0)))
    pad_idx = n + (jnp.arange(epad - e, dtype=jnp.int32) % (npad - n))
    src3 = jnp.concatenate([edge_index[0].astype(jnp.int32), pad_idx]
                           ).reshape(NW, chunks, LANE)
    dst3 = jnp.concatenate([edge_index[1].astype(jnp.int32), pad_idx]
                           ).reshape(NW, chunks, LANE)
    b1r = b1.reshape(1, d)
    b2r = b2.reshape(1, d2)

    histp = _sc_hist(dst3, npad, chunks)
    scaled1, dis = _tc_scale1(x_pad, W1, histp, npad, d)
    agg1 = _sc_agg(scaled1, src3, dst3, npad, chunks, d)
    scaled2 = _tc_l2(agg1, scaled1, dis, b1r, W2, npad, d, d2)
    agg2 = _sc_agg(scaled2, src3, dst3, npad, chunks, d)
    log_probs, logits = _tc_out(agg2, scaled2, dis, b2r, n, d, d2)
    return (log_probs, logits)


# speedup vs baseline: 2.8253x; 1.1167x over previous
import functools

import jax
import jax.numpy as jnp
from jax import lax
from jax.experimental import pallas as pl
from jax.experimental.pallas import tpu as pltpu
from jax.experimental.pallas import tpu_sc as plsc

NC = 2
NS = 16
NW = NC * NS
LANE = 128
RING = 2
HALVES = 2


def _tc_scale1(x_pad, W1, histp, npad, d):
    grid = npad // 256

    def body(x_ref, w_ref, hp_ref, s_ref, dis_ref):
        h = jnp.dot(x_ref[...], w_ref[...], preferred_element_type=jnp.float32)
        deg = 1.0 + hp_ref[0] + hp_ref[1]
        dis = lax.rsqrt(deg)
        dis_ref[...] = dis
        s_ref[...] = h * dis[:, 0:1]

    return pl.pallas_call(
        body,
        grid=(grid,),
        in_specs=[
            pl.BlockSpec((256, d), lambda r: (r, 0)),
            pl.BlockSpec((d, d), lambda r: (0, 0)),
            pl.BlockSpec((NC, 256, 16), lambda r: (0, r, 0)),
        ],
        out_specs=[
            pl.BlockSpec((256, d), lambda r: (r, 0)),
            pl.BlockSpec((256, 16), lambda r: (r, 0)),
        ],
        out_shape=[
            jax.ShapeDtypeStruct((npad, d), jnp.float32),
            jax.ShapeDtypeStruct((npad, 16), jnp.float32),
        ],
    )(x_pad, W1, histp)


def _sc_hist(dst3, npad, chunks):
    rows_per_tile = npad // NS
    zc = rows_per_tile // LANE
    mesh = plsc.VectorSubcoreMesh(core_axis_name="c", subcore_axis_name="s",
                                  num_cores=NC, num_subcores=NS)

    @functools.partial(
        pl.kernel,
        out_type=jax.ShapeDtypeStruct((NC, npad, 16), jnp.float32),
        mesh=mesh,
        scratch_types=[
            pltpu.VMEM((chunks, LANE), jnp.int32),
            pltpu.VMEM((LANE, 16), jnp.float32),
            pltpu.VMEM((LANE, 16), jnp.float32),
            pltpu.VMEM_SHARED((npad, 16), jnp.float32),
        ],
    )
    def hist(dst_hbm, out, dst_v, zb_v, ones_v, shared):
        c = lax.axis_index("c")
        s = lax.axis_index("s")
        wid = c * NS + s
        pltpu.sync_copy(dst_hbm.at[wid], dst_v)

        def init_body(i, _):
            zb_v[i, :] = jnp.zeros((16,), jnp.float32)
            ones_v[i, :] = jnp.ones((16,), jnp.float32)
            return _
        lax.fori_loop(0, LANE, init_body, None)
        for k in range(zc):
            pltpu.sync_copy(
                zb_v, shared.at[pl.ds(s * rows_per_tile + k * LANE, LANE)])
        plsc.subcore_barrier()

        def body(j, _):
            pltpu.sync_copy(ones_v, shared.at[dst_v.at[j]], add=True)
            return _
        lax.fori_loop(0, chunks, body, None)
        plsc.subcore_barrier()
        pltpu.sync_copy(shared.at[pl.ds(s * rows_per_tile, rows_per_tile)],
                        out.at[c, pl.ds(s * rows_per_tile, rows_per_tile)])

    return hist(dst3)


def _sc_agg(table, src3, dst3, npad, chunks, d):
    rows_per_tile = npad // NS
    zc = rows_per_tile // LANE
    mesh = plsc.VectorSubcoreMesh(core_axis_name="c", subcore_axis_name="s",
                                  num_cores=NC, num_subcores=NS)

    hchunks = chunks // HALVES

    @functools.partial(
        pl.kernel,
        out_type=jax.ShapeDtypeStruct((NC, npad, d), jnp.float32),
        mesh=mesh,
        scratch_types=[
            pltpu.VMEM((hchunks, LANE), jnp.int32),
            pltpu.VMEM((hchunks, LANE), jnp.int32),
            pltpu.VMEM((RING, LANE, d), jnp.float32),
            pltpu.VMEM_SHARED((npad, d), jnp.float32),
            pltpu.SemaphoreType.DMA,
            pltpu.SemaphoreType.DMA,
        ],
        compiler_params=pltpu.CompilerParams(use_tc_tiling_on_sc=False),
    )
    def agg(table_hbm, src_hbm, dst_hbm, out, src_v, dst_v, rows_v, shared,
            gsem0, gsem1):
        c = lax.axis_index("c")
        s = lax.axis_index("s")
        wid = c * NS + s

        def zero_body(i, _):
            for j in range(d // 16):
                rows_v[0, i, pl.ds(j * 16, 16)] = jnp.zeros((16,), jnp.float32)
            return _
        lax.fori_loop(0, LANE, zero_body, None)
        for k in range(zc):
            pltpu.sync_copy(rows_v.at[0],
                            shared.at[pl.ds(s * rows_per_tile + k * LANE,
                                            LANE)])
        plsc.subcore_barrier()

        gsems = [gsem0, gsem1]

        def half(h, _):
            pltpu.sync_copy(src_hbm.at[wid, pl.ds(h * hchunks, hchunks)],
                            src_v)
            pltpu.sync_copy(dst_hbm.at[wid, pl.ds(h * hchunks, hchunks)],
                            dst_v)

            def group(g, _):
                j = g * RING
                gds = [
                    pltpu.async_copy(table_hbm.at[src_v.at[j + b]],
                                     rows_v.at[b], gsems[b])
                    for b in range(RING)
                ]
                for b in range(RING):
                    gds[b].wait()
                    pltpu.sync_copy(rows_v.at[b], shared.at[dst_v.at[j + b]],
                                    add=True)
                return _
            lax.fori_loop(0, hchunks // RING, group, None)
            return _
        lax.fori_loop(0, HALVES, half, None)
        plsc.subcore_barrier()
        pltpu.sync_copy(shared.at[pl.ds(s * rows_per_tile, rows_per_tile)],
                        out.at[c, pl.ds(s * rows_per_tile, rows_per_tile)])

    return agg(table, src3, dst3)


def _tc_l2(agg1, scaled1, dis, b1, W2, npad, d, d2):
    grid = npad // 256

    def body(a_ref, s1_ref, dis_ref, b1_ref, w2_ref, o_ref):
        di = dis_ref[:, 0:1]
        u = di * (a_ref[0] + a_ref[1] + s1_ref[...]) + b1_ref[...]
        h = jnp.maximum(u, 0.0)
        o_ref[...] = di * jnp.dot(h, w2_ref[...],
                                  preferred_element_type=jnp.float32)

    return pl.pallas_call(
        body,
        grid=(grid,),
        in_specs=[
            pl.BlockSpec((NC, 256, d), lambda r: (0, r, 0)),
            pl.BlockSpec((256, d), lambda r: (r, 0)),
            pl.BlockSpec((256, 16), lambda r: (r, 0)),
            pl.BlockSpec((1, d), lambda r: (0, 0)),
            pl.BlockSpec((d, d2), lambda r: (0, 0)),
        ],
        out_specs=pl.BlockSpec((256, d2), lambda r: (r, 0)),
        out_shape=jax.ShapeDtypeStruct((npad, d2), jnp.float32),
    )(agg1, scaled1, dis, b1, W2)


def _tc_out(agg2, scaled2, dis, b2, n, d, d2):
    bn = next(b for b in (400, 256, 200, 128, 80, 16, 8) if n % b == 0)
    grid = n // bn

    def body(a_ref, s2_ref, dis_ref, b2_ref, lp_ref, lg_ref):
        di = dis_ref[:, 0:1]
        logits = di * (a_ref[0] + a_ref[1] + s2_ref[...]) + b2_ref[...]
        lg_ref[...] = logits
        m = jnp.max(logits, axis=1, keepdims=True)
        e = jnp.exp(logits - m)
        lse = m + jnp.log(jnp.sum(e, axis=1, keepdims=True))
        lp_ref[...] = logits - lse

    return pl.pallas_call(
        body,
        grid=(grid,),
        in_specs=[
            pl.BlockSpec((NC, bn, d2), lambda r: (0, r, 0)),
            pl.BlockSpec((bn, d2), lambda r: (r, 0)),
            pl.BlockSpec((bn, 16), lambda r: (r, 0)),
            pl.BlockSpec((1, d2), lambda r: (0, 0)),
        ],
        out_specs=[
            pl.BlockSpec((bn, d2), lambda r: (r, 0)),
            pl.BlockSpec((bn, d2), lambda r: (r, 0)),
        ],
        out_shape=[
            jax.ShapeDtypeStruct((n, d2), jnp.float32),
            jax.ShapeDtypeStruct((n, d2), jnp.float32),
        ],
    )(agg2, scaled2, dis, b2)


def kernel(x, edge_index, W1, b1, W2, b2):
    n, d = x.shape
    d2 = W2.shape[1]
    e = edge_index.shape[1]

    npad = -(-n // 2048) * 2048
    if npad == n:
        npad += 2048
    chunks = -(-e // (NW * LANE))
    gran = RING * HALVES
    chunks = -(-chunks // gran) * gran
    epad = NW * chunks * LANE

    x_pad = jnp.pad(x, ((0, npad - n), (0, 0)))
    pad_idx = n + (jnp.arange(epad - e, dtype=jnp.int32) % (npad - n))
    src3 = jnp.concatenate([edge_index[0].astype(jnp.int32), pad_idx]
                           ).reshape(NW, chunks, LANE)
    dst3 = jnp.concatenate([edge_index[1].astype(jnp.int32), pad_idx]
                           ).reshape(NW, chunks, LANE)
    b1r = b1.reshape(1, d)
    b2r = b2.reshape(1, d2)

    histp = _sc_hist(dst3, npad, chunks)
    scaled1, dis = _tc_scale1(x_pad, W1, histp, npad, d)
    agg1 = _sc_agg(scaled1, src3, dst3, npad, chunks, d)
    scaled2 = _tc_l2(agg1, scaled1, dis, b1r, W2, npad, d, d2)
    agg2 = _sc_agg(scaled2, src3, dst3, npad, chunks, d2)
    log_probs, logits = _tc_out(agg2, scaled2, dis, b2r, n, d, d2)
    return (log_probs, logits)
